# Initial kernel scaffold; baseline (speedup 1.0000x reference)
import functools

import jax
import jax.numpy as jnp
from jax import lax
from jax.experimental import pallas as pl
from jax.experimental.pallas import tpu as pltpu
from jax.experimental.pallas import tpu_sc as plsc

NC = 2
NS = 16
L = 16

G = 4
GROUP = 128


def _matmul_half_layout(x, W):
    N, DIN = x.shape
    DOUT = W.shape[0]
    H = DOUT // NC
    BM = 1000

    def body(x_ref, w_ref, o_ref):
        o_ref[...] = lax.dot_general(
            x_ref[...], w_ref[...],
            dimension_numbers=(((1,), (1,)), ((), ())),
            preferred_element_type=jnp.float32)

    return pl.pallas_call(
        body,
        grid=(NC, N // BM),
        in_specs=[
            pl.BlockSpec((BM, DIN), lambda c, m: (m, 0)),
            pl.BlockSpec((H, DIN), lambda c, m: (c, 0)),
        ],
        out_specs=pl.BlockSpec((BM, H), lambda c, m: (c * (N // BM) + m, 0)),
        out_shape=jax.ShapeDtypeStruct((NC * N, H), jnp.float32),
    )(x, W)


def _sc_spmm(h2, b2, srcg, dstg, wg, N, H):
    EG = srcg.shape[0]
    GPT = EG // NS
    NCHUNK = GPT // G
    CH = G * GROUP
    RPT = N // NS
    RB = 125
    NRB = RPT // RB

    mesh = plsc.VectorSubcoreMesh(
        core_axis_name="c", subcore_axis_name="s", num_cores=NC, num_subcores=NS)

    @functools.partial(
        pl.kernel,
        out_type=jax.ShapeDtypeStruct((N, NC * H), jnp.float32),
        mesh=mesh,
        scratch_types=[
            pltpu.VMEM((G, GROUP), jnp.int32),
            pltpu.VMEM((G, GROUP), jnp.int32),
            pltpu.VMEM((G, GROUP), jnp.float32),
            pltpu.VMEM((CH, H), jnp.float32),
            pltpu.VMEM((RB, H), jnp.float32),
            pltpu.VMEM((H,), jnp.float32),
            pltpu.VMEM_SHARED((N, H), jnp.float32),
            pltpu.SemaphoreType.DMA,
        ],
    )
    def spmm(h2_hbm, b2_hbm, srcg_hbm, dstg_hbm, wg_hbm, out_hbm,
             src_v, dst_v, w_v, rows_v, stage_v, b_v, acc, sem):
        c = lax.axis_index("c")
        s = lax.axis_index("s")

        pltpu.sync_copy(b2_hbm.at[c], b_v)

        def fill_row(r, _):
            for d in range(H // L):
                sl = pl.ds(d * L, L)
                stage_v[r, sl] = b_v[sl]
            return 0
        lax.fori_loop(0, RB, fill_row, 0)

        def init_blk(i, _):
            pltpu.sync_copy(stage_v, acc.at[pl.ds(s * RPT + i * RB, RB)])
            return 0
        lax.fori_loop(0, NRB, init_blk, 0)
        plsc.subcore_barrier()

        coff = jnp.full((L,), c * N, dtype=jnp.int32)

        def chunk(ci, _):
            g0 = s * GPT + ci * G
            pltpu.sync_copy(srcg_hbm.at[pl.ds(g0, G)], src_v)
            pltpu.sync_copy(dstg_hbm.at[pl.ds(g0, G)], dst_v)
            pltpu.sync_copy(wg_hbm.at[pl.ds(g0, G)], w_v)
            for g in range(G):
                for j in range(GROUP // L):
                    sl = pl.ds(j * L, L)
                    src_v[g, sl] = src_v[g, sl] + coff
            cps = [
                pltpu.async_copy(h2_hbm.at[src_v.at[g]],
                                 rows_v.at[pl.ds(g * GROUP, GROUP)], sem)
                for g in range(G)
            ]
            for cp in cps:
                cp.wait()
            for g in range(G):
                def scale(k, _, g=g):
                    ws = w_v[g, k]
                    row = g * GROUP + k
                    for d in range(H // L):
                        sl = pl.ds(d * L, L)
                        rows_v[row, sl] = rows_v[row, sl] * ws
                    return 0
                lax.fori_loop(0, GROUP, scale, 0)
                pltpu.sync_copy(rows_v.at[pl.ds(g * GROUP, GROUP)],
                                acc.at[dst_v.at[g]], add=True)
            return 0
        lax.fori_loop(0, NCHUNK, chunk, 0)
        plsc.subcore_barrier()

        def outblk(i, _):
            r0 = s * RPT + i * RB
            pltpu.sync_copy(acc.at[pl.ds(r0, RB)], stage_v)
            pltpu.sync_copy(stage_v, out_hbm.at[pl.ds(r0, RB), pl.ds(c * H, H)])
            return 0
        lax.fori_loop(0, NRB, outblk, 0)

    return spmm(h2, b2, srcg, dstg, wg)


def kernel(input, edge_index, edge_weight, W, b):
    x = input
    N = x.shape[0]
    DOUT = W.shape[0]
    H = DOUT // NC

    dst = edge_index[0].astype(jnp.int32)
    src = edge_index[1].astype(jnp.int32)
    w = edge_weight.astype(jnp.float32)
    E = src.shape[0]

    EG = -(-E // GROUP)
    EG = -(-EG // NS) * NS
    pad = EG * GROUP - E
    srcg = jnp.pad(src, (0, pad)).reshape(EG, GROUP)
    dstg = jnp.pad(dst, (0, pad)).reshape(EG, GROUP)
    wg = jnp.pad(w, (0, pad)).reshape(EG, GROUP)
    b2 = b.astype(jnp.float32).reshape(NC, H)

    h2 = _matmul_half_layout(x, W)
    return _sc_spmm(h2, b2, srcg, dstg, wg, N, H)


# speedup vs baseline: 3.0180x; 3.0180x over previous
import functools

import jax
import jax.numpy as jnp
from jax import lax
from jax.experimental import pallas as pl
from jax.experimental.pallas import tpu as pltpu
from jax.experimental.pallas import tpu_sc as plsc

NC = 2
NS = 16
L = 16

G = 2
GROUP = 128


def _matmul_half_layout(x, W):
    N, DIN = x.shape
    DOUT = W.shape[0]
    H = DOUT // NC
    BM = 1000

    def body(x_ref, w_ref, o_ref):
        o_ref[...] = lax.dot_general(
            x_ref[...], w_ref[...],
            dimension_numbers=(((1,), (1,)), ((), ())),
            preferred_element_type=jnp.float32)

    return pl.pallas_call(
        body,
        grid=(NC, N // BM),
        in_specs=[
            pl.BlockSpec((BM, DIN), lambda c, m: (m, 0)),
            pl.BlockSpec((H, DIN), lambda c, m: (c, 0)),
        ],
        out_specs=pl.BlockSpec((BM, H), lambda c, m: (c * (N // BM) + m, 0)),
        out_shape=jax.ShapeDtypeStruct((NC * N, H), jnp.float32),
    )(x, W)


def _sc_spmm(h2, b3, srcg, dstg, wg, N, H):
    EG = srcg.shape[0]
    SUP = 8
    GPT = EG // NS
    NSUP = GPT // SUP
    RB = 80
    NBLK = N // RB
    BPT = -(-NBLK // NS)

    mesh = plsc.VectorSubcoreMesh(
        core_axis_name="c", subcore_axis_name="s", num_cores=NC, num_subcores=NS)

    @functools.partial(
        pl.kernel,
        out_type=jax.ShapeDtypeStruct((N, NC * H), jnp.float32),
        mesh=mesh,
        scratch_types=[
            pltpu.VMEM((SUP, GROUP), jnp.int32),
            pltpu.VMEM((SUP, GROUP), jnp.int32),
            pltpu.VMEM((SUP, GROUP), jnp.float32),
            pltpu.VMEM((G * GROUP, H), jnp.float32),
            pltpu.VMEM((RB, H), jnp.float32),
            pltpu.VMEM((1, H), jnp.float32),
            pltpu.VMEM_SHARED((N, H), jnp.float32),
            pltpu.SemaphoreType.DMA,
        ],
    )
    def spmm(h2_hbm, b3_hbm, srcg_hbm, dstg_hbm, wg_hbm, out_hbm,
             src_v, dst_v, w_v, rows_v, stage_v, b_v, acc, sem):
        c = lax.axis_index("c")
        s = lax.axis_index("s")

        pltpu.sync_copy(b3_hbm.at[c], b_v)

        def fill_row(r, _):
            for d in range(H // L):
                sl = pl.ds(d * L, L)
                stage_v[r, sl] = b_v[0, sl]
            return 0
        lax.fori_loop(0, RB, fill_row, 0)

        def init_blk(i, _):
            blk = s + i * NS

            @pl.when(blk < NBLK)
            def _():
                pltpu.sync_copy(stage_v, acc.at[pl.ds(blk * RB, RB)])
            return 0
        lax.fori_loop(0, BPT, init_blk, 0)
        plsc.subcore_barrier()

        coff = jnp.full((L,), c * N, dtype=jnp.int32)

        def superchunk(ci, _):
            g0 = s * GPT + ci * SUP
            pltpu.sync_copy(srcg_hbm.at[pl.ds(g0, SUP)], src_v)
            pltpu.sync_copy(dstg_hbm.at[pl.ds(g0, SUP)], dst_v)
            pltpu.sync_copy(wg_hbm.at[pl.ds(g0, SUP)], w_v)
            for g in range(SUP):
                for j in range(GROUP // L):
                    sl = pl.ds(j * L, L)
                    src_v[g, sl] = src_v[g, sl] + coff
            for h in range(SUP // G):
                cps = [
                    pltpu.async_copy(h2_hbm.at[src_v.at[h * G + g]],
                                     rows_v.at[pl.ds(g * GROUP, GROUP)], sem)
                    for g in range(G)
                ]
                for cp in cps:
                    cp.wait()
                for g in range(G):
                    def scale16(j, _, g=g, h=h):
                        wv = w_v[h * G + g, pl.ds(j * L, L)]
                        base = j * L
                        for t in range(L):
                            ws = wv[t]
                            row = g * GROUP + base + t
                            for d in range(H // L):
                                sl = pl.ds(d * L, L)
                                rows_v[row, sl] = rows_v[row, sl] * ws
                        return 0
                    lax.fori_loop(0, GROUP // L, scale16, 0)
                    pltpu.sync_copy(rows_v.at[pl.ds(g * GROUP, GROUP)],
                                    acc.at[dst_v.at[h * G + g]], add=True)
            return 0
        lax.fori_loop(0, NSUP, superchunk, 0)
        plsc.subcore_barrier()

        def outblk(i, _):
            blk = s + i * NS

            @pl.when(blk < NBLK)
            def _():
                pltpu.sync_copy(acc.at[pl.ds(blk * RB, RB)], stage_v)
                pltpu.sync_copy(stage_v,
                                out_hbm.at[pl.ds(blk * RB, RB), pl.ds(c * H, H)])
            return 0
        lax.fori_loop(0, BPT, outblk, 0)

    return spmm(h2, b3, srcg, dstg, wg)


def kernel(input, edge_index, edge_weight, W, b):
    x = input
    N = x.shape[0]
    DOUT = W.shape[0]
    H = DOUT // NC

    dst = edge_index[0].astype(jnp.int32)
    src = edge_index[1].astype(jnp.int32)
    w = edge_weight.astype(jnp.float32)
    E = src.shape[0]

    EG = -(-E // (GROUP * NS * 8)) * (NS * 8)
    pad = EG * GROUP - E
    srcg = jnp.pad(src, (0, pad)).reshape(EG, GROUP)
    dstg = jnp.pad(dst, (0, pad)).reshape(EG, GROUP)
    wg = jnp.pad(w, (0, pad)).reshape(EG, GROUP)
    b3 = b.astype(jnp.float32).reshape(NC, 1, H)

    h2 = _matmul_half_layout(x, W)
    return _sc_spmm(h2, b3, srcg, dstg, wg, N, H)


# speedup vs baseline: 3.5217x; 1.1669x over previous
import functools

import jax
import jax.numpy as jnp
from jax import lax
from jax.experimental import pallas as pl
from jax.experimental.pallas import tpu as pltpu
from jax.experimental.pallas import tpu_sc as plsc

NC = 2
NS = 16
GROUP = 128
SUP = 8


def _matmul_half_layout(x, W):
    N, DIN = x.shape
    DOUT = W.shape[0]
    H = DOUT // NC
    BM = 1000

    def body(x_ref, w_ref, o_ref):
        o_ref[...] = lax.dot_general(
            x_ref[...], w_ref[...],
            dimension_numbers=(((1,), (1,)), ((), ())),
            preferred_element_type=jnp.float32)

    return pl.pallas_call(
        body,
        grid=(NC, N // BM),
        in_specs=[
            pl.BlockSpec((BM, DIN), lambda c, m: (m, 0)),
            pl.BlockSpec((H, DIN), lambda c, m: (c, 0)),
        ],
        out_specs=pl.BlockSpec((BM, H), lambda c, m: (c * (N // BM) + m, 0)),
        out_shape=jax.ShapeDtypeStruct((NC * N, H), jnp.float32),
    )(x, W)


def _sc_spmm(h2, b3, srcg, dstg, wg, N, H):
    EG = srcg.shape[0]
    GPT = EG // NS
    NSUP = GPT // SUP
    RB = 80
    NBLK = N // RB
    BPT = -(-NBLK // NS)

    mesh = plsc.VectorSubcoreMesh(
        core_axis_name="c", subcore_axis_name="s", num_cores=NC, num_subcores=NS)

    @functools.partial(
        pl.kernel,
        out_type=jax.ShapeDtypeStruct((N, NC * H), jnp.float32),
        mesh=mesh,
        scratch_types=[
            pltpu.VMEM((SUP, GROUP), jnp.int32),
            pltpu.VMEM((SUP, GROUP), jnp.int32),
            pltpu.VMEM((SUP, GROUP), jnp.float32),
            pltpu.VMEM((GROUP, H), jnp.float32),
            pltpu.VMEM((GROUP, H), jnp.float32),
            pltpu.VMEM((1, H), jnp.float32),
            pltpu.VMEM_SHARED((N, H), jnp.float32),
            pltpu.SemaphoreType.DMA,
            pltpu.SemaphoreType.DMA,
            pltpu.SemaphoreType.DMA,
            pltpu.SemaphoreType.DMA,
        ],
    )
    def spmm(h2_hbm, b3_hbm, srcg_hbm, dstg_hbm, wg_hbm, out_hbm,
             src_v, dst_v, w_v, buf0, buf1, b_v, acc,
             sem_g0, sem_g1, sem_s0, sem_s1):
        c = lax.axis_index("c")
        s = lax.axis_index("s")
        bufs = (buf0, buf1)
        gsems = (sem_g0, sem_g1)
        ssems = (sem_s0, sem_s1)

        pltpu.sync_copy(b3_hbm.at[c], b_v)

        def fill_row(r, _):
            for d in range(H // 16):
                sl = pl.ds(d * 16, 16)
                buf0[r, sl] = b_v[0, sl]
            return 0
        lax.fori_loop(0, RB, fill_row, 0)

        def init_blk(i, _):
            blk = s + i * NS

            @pl.when(blk < NBLK)
            def _():
                pltpu.sync_copy(buf0.at[pl.ds(0, RB)], acc.at[pl.ds(blk * RB, RB)])
            return 0
        lax.fori_loop(0, BPT, init_blk, 0)
        plsc.subcore_barrier()

        coff = jnp.full((16,), c * N, dtype=jnp.int32)

        def scale_group(p, q):
            buf = bufs[p]

            def body16(j, _):
                wv = w_v[q, pl.ds(j * 16, 16)]
                for t in range(16):
                    ws = wv[t]
                    row = j * 16 + t
                    for d in range(H // 16):
                        sl = pl.ds(d * 16, 16)
                        buf[row, sl] = buf[row, sl] * ws
                return 0
            lax.fori_loop(0, GROUP // 16, body16, 0)

        def fire_gather(p, q):
            return pltpu.async_copy(h2_hbm.at[src_v.at[q]], bufs[p], gsems[p])

        def superchunk(ci, _):
            g0 = s * GPT + ci * SUP
            pltpu.sync_copy(srcg_hbm.at[pl.ds(g0, SUP)], src_v)
            pltpu.sync_copy(dstg_hbm.at[pl.ds(g0, SUP)], dst_v)
            pltpu.sync_copy(wg_hbm.at[pl.ds(g0, SUP)], w_v)
            def add_off(g, _):
                for j in range(GROUP // 16):
                    sl = pl.ds(j * 16, 16)
                    src_v[g, sl] = src_v[g, sl] + coff
                return 0
            lax.fori_loop(0, SUP, add_off, 0)

            gat = {0: None, 1: None}
            gat[0] = fire_gather(0, 0)
            for q in range(SUP):
                p = q % 2
                gat[p].wait()
                if q + 1 < SUP:
                    gat[1 - p] = fire_gather(1 - p, q + 1)
                scale_group(p, q)
                pltpu.sync_copy(bufs[p], acc.at[dst_v.at[q]], add=True)
            return 0
        lax.fori_loop(0, NSUP, superchunk, 0)
        plsc.subcore_barrier()

        def outblk(i, _):
            blk = s + i * NS

            @pl.when(blk < NBLK)
            def _():
                pltpu.sync_copy(acc.at[pl.ds(blk * RB, RB)], buf0.at[pl.ds(0, RB)])
                pltpu.sync_copy(buf0.at[pl.ds(0, RB)],
                                out_hbm.at[pl.ds(blk * RB, RB), pl.ds(c * H, H)])
            return 0
        lax.fori_loop(0, BPT, outblk, 0)

    return spmm(h2, b3, srcg, dstg, wg)


def kernel(input, edge_index, edge_weight, W, b):
    x = input
    N = x.shape[0]
    DOUT = W.shape[0]
    H = DOUT // NC

    dst = edge_index[0].astype(jnp.int32)
    src = edge_index[1].astype(jnp.int32)
    w = edge_weight.astype(jnp.float32)
    E = src.shape[0]

    EG = -(-E // (GROUP * NS * SUP)) * (NS * SUP)
    pad = EG * GROUP - E
    srcg = jnp.pad(src, (0, pad)).reshape(EG, GROUP)
    dstg = jnp.pad(dst, (0, pad)).reshape(EG, GROUP)
    wg = jnp.pad(w, (0, pad)).reshape(EG, GROUP)
    b3 = b.astype(jnp.float32).reshape(NC, 1, H)

    h2 = _matmul_half_layout(x, W)
    return _sc_spmm(h2, b3, srcg, dstg, wg, N, H)


# speedup vs baseline: 3.5256x; 1.0011x over previous
import functools

import jax
import jax.numpy as jnp
from jax import lax
from jax.experimental import pallas as pl
from jax.experimental.pallas import tpu as pltpu
from jax.experimental.pallas import tpu_sc as plsc

NC = 2
NS = 16
GROUP = 128
SUP = 8


def _matmul_half_layout(x, W):
    N, DIN = x.shape
    DOUT = W.shape[0]
    H = DOUT // NC
    BM = 1000

    def body(x_ref, w_ref, o_ref):
        o_ref[...] = lax.dot_general(
            x_ref[...], w_ref[...],
            dimension_numbers=(((1,), (1,)), ((), ())),
            preferred_element_type=jnp.float32)

    return pl.pallas_call(
        body,
        grid=(NC, N // BM),
        in_specs=[
            pl.BlockSpec((BM, DIN), lambda c, m: (m, 0)),
            pl.BlockSpec((H, DIN), lambda c, m: (c, 0)),
        ],
        out_specs=pl.BlockSpec((BM, H), lambda c, m: (c * (N // BM) + m, 0)),
        out_shape=jax.ShapeDtypeStruct((NC * N, H), jnp.float32),
    )(x, W)


def _sc_spmm(h2, b3, srcg, dstg, wg, N, H):
    EG = srcg.shape[0]
    GPT = EG // NS
    NSUP = GPT // SUP
    RB = 80
    NBLK = N // RB
    BPT = -(-NBLK // NS)

    mesh = plsc.VectorSubcoreMesh(
        core_axis_name="c", subcore_axis_name="s", num_cores=NC, num_subcores=NS)

    @functools.partial(
        pl.kernel,
        out_type=jax.ShapeDtypeStruct((N, NC * H), jnp.float32),
        mesh=mesh,
        scratch_types=[
            pltpu.VMEM((SUP, GROUP), jnp.int32),
            pltpu.VMEM((SUP, GROUP), jnp.int32),
            pltpu.VMEM((SUP, GROUP), jnp.float32),
            pltpu.VMEM((GROUP, H), jnp.float32),
            pltpu.VMEM((GROUP, H), jnp.float32),
            pltpu.VMEM((1, H), jnp.float32),
            pltpu.VMEM_SHARED((N, H), jnp.float32),
            pltpu.SemaphoreType.DMA,
            pltpu.SemaphoreType.DMA,
            pltpu.SemaphoreType.DMA,
            pltpu.SemaphoreType.DMA,
        ],
    )
    def spmm(h2_hbm, b3_hbm, srcg_hbm, dstg_hbm, wg_hbm, out_hbm,
             src_v, dst_v, w_v, buf0, buf1, b_v, acc,
             sem_g0, sem_g1, sem_s0, sem_s1):
        c = lax.axis_index("c")
        s = lax.axis_index("s")
        bufs = (buf0, buf1)
        gsems = (sem_g0, sem_g1)
        ssems = (sem_s0, sem_s1)

        pltpu.sync_copy(b3_hbm.at[c], b_v)

        def fill_row(r, _):
            for d in range(H // 16):
                sl = pl.ds(d * 16, 16)
                buf0[r, sl] = b_v[0, sl]
            return 0
        lax.fori_loop(0, RB, fill_row, 0)

        def init_blk(i, _):
            blk = s + i * NS

            @pl.when(blk < NBLK)
            def _():
                pltpu.sync_copy(buf0.at[pl.ds(0, RB)], acc.at[pl.ds(blk * RB, RB)])
            return 0
        lax.fori_loop(0, BPT, init_blk, 0)
        plsc.subcore_barrier()

        def scale_group(p, q):
            buf = bufs[p]

            @plsc.parallel_loop(0, GROUP // 16, 1, unroll=2)
            def body16(j):
                wv = w_v[q, pl.ds(j * 16, 16)]
                for t in range(16):
                    ws = wv[t]
                    row = j * 16 + t
                    for d in range(H // 16):
                        sl = pl.ds(d * 16, 16)
                        buf[row, sl] = buf[row, sl] * ws

        h2c = h2_hbm.at[pl.ds(pl.multiple_of(c * N, 8), N)]

        def fire_gather(p, q):
            return pltpu.async_copy(h2c.at[src_v.at[q]], bufs[p], gsems[p])

        def superchunk(ci, _):
            g0 = s * GPT + ci * SUP
            pltpu.sync_copy(srcg_hbm.at[pl.ds(g0, SUP)], src_v)
            pltpu.sync_copy(dstg_hbm.at[pl.ds(g0, SUP)], dst_v)
            pltpu.sync_copy(wg_hbm.at[pl.ds(g0, SUP)], w_v)
            gat = {0: None, 1: None}
            gat[0] = fire_gather(0, 0)
            for q in range(SUP):
                p = q % 2
                gat[p].wait()
                if q + 1 < SUP:
                    gat[1 - p] = fire_gather(1 - p, q + 1)
                scale_group(p, q)
                pltpu.sync_copy(bufs[p], acc.at[dst_v.at[q]], add=True)
            return 0
        lax.fori_loop(0, NSUP, superchunk, 0)
        plsc.subcore_barrier()

        def outblk(i, _):
            blk = s + i * NS

            @pl.when(blk < NBLK)
            def _():
                pltpu.sync_copy(acc.at[pl.ds(blk * RB, RB)], buf0.at[pl.ds(0, RB)])
                pltpu.sync_copy(buf0.at[pl.ds(0, RB)],
                                out_hbm.at[pl.ds(blk * RB, RB), pl.ds(c * H, H)])
            return 0
        lax.fori_loop(0, BPT, outblk, 0)

    return spmm(h2, b3, srcg, dstg, wg)


def kernel(input, edge_index, edge_weight, W, b):
    x = input
    N = x.shape[0]
    DOUT = W.shape[0]
    H = DOUT // NC

    dst = edge_index[0].astype(jnp.int32)
    src = edge_index[1].astype(jnp.int32)
    w = edge_weight.astype(jnp.float32)
    E = src.shape[0]

    EG = -(-E // (GROUP * NS * SUP)) * (NS * SUP)
    pad = EG * GROUP - E
    srcg = jnp.pad(src, (0, pad)).reshape(EG, GROUP)
    dstg = jnp.pad(dst, (0, pad)).reshape(EG, GROUP)
    wg = jnp.pad(w, (0, pad)).reshape(EG, GROUP)
    b3 = b.astype(jnp.float32).reshape(NC, 1, H)

    h2 = _matmul_half_layout(x, W)
    return _sc_spmm(h2, b3, srcg, dstg, wg, N, H)
